# speedup vs baseline: 2.4593x; 2.4593x over previous
import functools

import jax
import jax.numpy as jnp
from jax.experimental import pallas as pl

SEQ = 81
VOCAB = 10
HID = 512
BB = 64


def _body(x_ref, tok_ref, pos_ref, out_ref):
    x = x_ref[...]
    acc = jnp.broadcast_to(pos_ref[...][None, :, :], (BB, SEQ, HID))
    for v in range(VOCAB):
        m = (x == v).astype(jnp.float32)[:, :, None]
        acc = acc + m * tok_ref[v, :][None, None, :]
    out_ref[...] = acc


def kernel(x, token_emb, pos_emb):
    B = x.shape[0]
    grid = (B // BB,)
    out = pl.pallas_call(
        _body,
        grid=grid,
        in_specs=[
            pl.BlockSpec((BB, SEQ), lambda i: (i, 0)),
            pl.BlockSpec((VOCAB, HID), lambda i: (0, 0)),
            pl.BlockSpec((SEQ, HID), lambda i: (0, 0)),
        ],
        out_specs=pl.BlockSpec((BB, SEQ, HID), lambda i: (i, 0, 0)),
        out_shape=jax.ShapeDtypeStruct((B, SEQ, HID), jnp.float32),
    )(x, token_emb, pos_emb)
    return out


# speedup vs baseline: 2.6156x; 1.0636x over previous
import functools

import jax
import jax.numpy as jnp
from jax.experimental import pallas as pl

SEQ = 81
VOCAB = 10
VPAD = 16
HID = 512
BB = 128


def _body(x_ref, tok_ref, pos_ref, out_ref):
    xb = jnp.broadcast_to(x_ref[...][:, :, None], (BB, SEQ, HID))
    acc = jnp.broadcast_to(pos_ref[...][None, :, :], (BB, SEQ, HID))
    for v in range(VOCAB):
        tv = jnp.broadcast_to(tok_ref[v, :][None, None, :], (BB, SEQ, HID))
        acc = acc + jnp.where(xb == v, tv, 0.0)
    out_ref[...] = acc


def kernel(x, token_emb, pos_emb):
    B = x.shape[0]
    tokp = jnp.pad(token_emb, ((0, VPAD - VOCAB), (0, 0)))
    grid = (B // BB,)
    out = pl.pallas_call(
        _body,
        grid=grid,
        in_specs=[
            pl.BlockSpec((BB, SEQ), lambda i: (i, 0)),
            pl.BlockSpec((VPAD, HID), lambda i: (0, 0)),
            pl.BlockSpec((SEQ, HID), lambda i: (0, 0)),
        ],
        out_specs=pl.BlockSpec((BB, SEQ, HID), lambda i: (i, 0, 0)),
        out_shape=jax.ShapeDtypeStruct((B, SEQ, HID), jnp.float32),
    )(x, tokp, pos_emb)
    return out


# speedup vs baseline: 3.3676x; 1.2875x over previous
import functools

import jax
import jax.numpy as jnp
from jax import lax
from jax.experimental import pallas as pl
from jax.experimental.pallas import tpu as pltpu

SEQ = 81
VOCAB = 10
HID = 512
NB = 64
NBUF = 4


def _compute(x, tok, pos):
    shape = (NB, SEQ, HID)
    xb = jnp.broadcast_to(x[:, :, None], shape)

    def tv(v):
        return jnp.broadcast_to(tok[v, :][None, None, :], shape)

    m0 = (xb & 1) != 0
    m1 = (xb & 2) != 0
    m2 = (xb & 4) != 0
    m3 = (xb & 8) != 0
    t01 = jnp.where(m0, tv(1), tv(0))
    t23 = jnp.where(m0, tv(3), tv(2))
    t45 = jnp.where(m0, tv(5), tv(4))
    t67 = jnp.where(m0, tv(7), tv(6))
    t89 = jnp.where(m0, tv(9), tv(8))
    t03 = jnp.where(m1, t23, t01)
    t47 = jnp.where(m1, t67, t45)
    t07 = jnp.where(m2, t47, t03)
    tok_sel = jnp.where(m3, t89, t07)
    return tok_sel + jnp.broadcast_to(pos[None, :, :], shape)


def _body(x_hbm, tok_ref, pos_ref, out_hbm, x_all, bufs, in_sem, out_sems):
    nblk = x_hbm.shape[0] // NB
    pltpu.make_async_copy(x_hbm, x_all, in_sem).start()
    pltpu.make_async_copy(x_hbm, x_all, in_sem).wait()
    tok = tok_ref[...]
    pos = pos_ref[...]

    def step(i, _):
        slot = lax.rem(i, NBUF)

        @pl.when(i >= NBUF)
        def _wait_prev():
            prev = i - NBUF
            pltpu.make_async_copy(
                bufs.at[slot],
                out_hbm.at[pl.ds(prev * NB, NB)],
                out_sems.at[slot],
            ).wait()

        x = x_all[pl.ds(i * NB, NB), :]
        bufs[slot] = _compute(x, tok, pos)
        pltpu.make_async_copy(
            bufs.at[slot],
            out_hbm.at[pl.ds(i * NB, NB)],
            out_sems.at[slot],
        ).start()
        return 0

    lax.fori_loop(0, nblk, step, 0)

    def drain(k, _):
        slot = lax.rem(nblk - NBUF + k, NBUF)
        pltpu.make_async_copy(
            bufs.at[slot],
            out_hbm.at[pl.ds((nblk - NBUF + k) * NB, NB)],
            out_sems.at[slot],
        ).wait()
        return 0

    lax.fori_loop(0, NBUF, drain, 0)


def kernel(x, token_emb, pos_emb):
    B = x.shape[0]
    out = pl.pallas_call(
        _body,
        in_specs=[
            pl.BlockSpec(memory_space=pl.ANY),
            pl.BlockSpec(memory_space=pltpu.MemorySpace.VMEM),
            pl.BlockSpec(memory_space=pltpu.MemorySpace.VMEM),
        ],
        out_specs=pl.BlockSpec(memory_space=pl.ANY),
        out_shape=jax.ShapeDtypeStruct((B, SEQ, HID), jnp.float32),
        scratch_shapes=[
            pltpu.VMEM((B, SEQ), jnp.int32),
            pltpu.VMEM((NBUF, NB, SEQ, HID), jnp.float32),
            pltpu.SemaphoreType.DMA,
            pltpu.SemaphoreType.DMA((NBUF,)),
        ],
    )(x, token_emb, pos_emb)
    return out
